# Initial kernel scaffold; baseline (speedup 1.0000x reference)
import jax
import jax.numpy as jnp
from jax.experimental import pallas as pl


def kernel(tokens, attn_mask, edu_reps, speaker_ids, speaker_memory, Win, b_in, Wout, b_out, ffn_W1, ffn_b1, ffn_W2, ffn_b2, ln1_g, ln1_b, ln2_g, ln2_b, Wsq, Wsk, Wcq, Wck, Wih, Whh, bih, bhh):
    raise NotImplementedError("write your pallas kernel here")


# speedup vs baseline: 3.4352x; 3.4352x over previous
import functools

import jax
import jax.numpy as jnp
from jax.experimental import pallas as pl
from jax.experimental.pallas import tpu as pltpu

N_HEADS = 12
TOP_K = 5
EPS = 1e-5
NEG = -1e30


def _ln(x, g, b):
    mu = jnp.mean(x, axis=-1, keepdims=True)
    var = jnp.mean((x - mu) ** 2, axis=-1, keepdims=True)
    return (x - mu) / jnp.sqrt(var + EPS) * g + b


def _sigmoid(x):
    return 1.0 / (1.0 + jnp.exp(-x))


def _transformer_kernel(x_ref, WinT_ref, b_in_ref, WoutT_ref, b_out_ref,
                        W1T_ref, b1_ref, W2T_ref, b2_ref,
                        ln1_g_ref, ln1_b_ref, ln2_g_ref, ln2_b_ref, o_ref):
    x = x_ref[0]
    L, D = x.shape
    dh = D // N_HEADS
    qkv = jnp.dot(x, WinT_ref[...], preferred_element_type=jnp.float32)
    qkv = qkv + b_in_ref[...]
    heads = []
    scale = 1.0 / (dh ** 0.5)
    for h in range(N_HEADS):
        q = qkv[:, h * dh:(h + 1) * dh]
        k = qkv[:, D + h * dh:D + (h + 1) * dh]
        v = qkv[:, 2 * D + h * dh:2 * D + (h + 1) * dh]
        s = jax.lax.dot_general(q, k, (((1,), (1,)), ((), ())),
                                preferred_element_type=jnp.float32) * scale
        m = jnp.max(s, axis=-1, keepdims=True)
        e = jnp.exp(s - m)
        p = e / jnp.sum(e, axis=-1, keepdims=True)
        heads.append(jnp.dot(p, v, preferred_element_type=jnp.float32))
    attn = jnp.concatenate(heads, axis=1)
    attn = jnp.dot(attn, WoutT_ref[...], preferred_element_type=jnp.float32)
    attn = attn + b_out_ref[...]
    tv = _ln(x + attn, ln1_g_ref[...], ln1_b_ref[...])
    h1 = jnp.dot(tv, W1T_ref[...], preferred_element_type=jnp.float32)
    h1 = h1 + b1_ref[...]
    h1 = h1 * 0.5 * (1.0 + jax.lax.erf(h1 * (2.0 ** -0.5)))
    h2 = jnp.dot(h1, W2T_ref[...], preferred_element_type=jnp.float32)
    h2 = h2 + b2_ref[...]
    o_ref[0] = _ln(tv + h2, ln2_g_ref[...], ln2_b_ref[...])


def _memory_kernel(ids_ref, edu_ref, mem_ref,
                   WsqT_ref, WskT_ref, WcqT_ref, WckT_ref,
                   WihT_ref, WhhT_ref, bih_ref, bhh_ref,
                   omem_ref, gi_ref,
                   *, B, T, S, D):
    edu = edu_ref[...]
    Ks = jnp.dot(edu, WskT_ref[...], preferred_element_type=jnp.float32)
    Qs = jnp.dot(edu, WsqT_ref[...], preferred_element_type=jnp.float32)
    Kc = jnp.dot(edu, WckT_ref[...], preferred_element_type=jnp.float32)
    Qc = jnp.dot(edu, WcqT_ref[...], preferred_element_type=jnp.float32)

    jidx = jax.lax.broadcasted_iota(jnp.int32, (T, T), 1)
    tidx = jax.lax.broadcasted_iota(jnp.int32, (T, T), 0)
    kk = jnp.minimum(jnp.float32(TOP_K), tidx[:, :1].astype(jnp.float32))
    kk_div = jnp.maximum(kk, 1.0)

    summaries = []
    for b in range(B):
        ks = Ks[b * T:(b + 1) * T]
        qs = Qs[b * T:(b + 1) * T]
        kc = Kc[b * T:(b + 1) * T]
        qc = Qc[b * T:(b + 1) * T]
        sc = (jax.lax.dot_general(qs, ks, (((1,), (1,)), ((), ())),
                                  preferred_element_type=jnp.float32)
              + jax.lax.dot_general(qc, kc, (((1,), (1,)), ((), ())),
                                    preferred_element_type=jnp.float32))
        cur = jnp.where(jidx < tidx, sc, NEG)
        w = jnp.zeros((T, T), jnp.float32)
        for i in range(TOP_K):
            m = jnp.max(cur, axis=1, keepdims=True)
            eq = cur == m
            jsel = jnp.min(jnp.where(eq, jidx, T), axis=1, keepdims=True)
            first = jidx == jsel
            take = first & (jnp.float32(i) < kk)
            w = w + jnp.where(take, 1.0, 0.0)
            cur = jnp.where(first, NEG, cur)
        summaries.append(jnp.dot(w, edu[b * T:(b + 1) * T],
                                 preferred_element_type=jnp.float32) / kk_div)
    summ = jnp.concatenate(summaries, axis=0)
    gi = jnp.dot(summ, WihT_ref[...], preferred_element_type=jnp.float32)
    gi_ref[...] = gi + bih_ref[...]

    omem_ref[...] = mem_ref[...]

    WhhT = WhhT_ref[...]
    bhh = bhh_ref[...]

    def step(t, _):
        hs = []
        for b in range(B):
            s = ids_ref[b, t]
            hs.append(omem_ref[pl.ds(b * S + s, 1), :])
        H = jnp.concatenate(hs, axis=0)
        gh = jnp.dot(H, WhhT, preferred_element_type=jnp.float32) + bhh
        gis = []
        for b in range(B):
            gis.append(gi_ref[pl.ds(b * T + t, 1), :])
        gib = jnp.concatenate(gis, axis=0)
        r = _sigmoid(gib[:, :D] + gh[:, :D])
        z = _sigmoid(gib[:, D:2 * D] + gh[:, D:2 * D])
        n = jnp.tanh(gib[:, 2 * D:] + r * gh[:, 2 * D:])
        newH = (1.0 - z) * n + z * H
        for b in range(B):
            s = ids_ref[b, t]
            omem_ref[pl.ds(b * S + s, 1), :] = newH[b:b + 1, :]
        return 0

    jax.lax.fori_loop(1, T, step, 0)


def kernel(tokens, attn_mask, edu_reps, speaker_ids, speaker_memory,
           Win, b_in, Wout, b_out, ffn_W1, ffn_b1, ffn_W2, ffn_b2,
           ln1_g, ln1_b, ln2_g, ln2_b, Wsq, Wsk, Wcq, Wck,
           Wih, Whh, bih, bhh):
    B, T, L, D = tokens.shape
    S = speaker_memory.shape[1]
    FF = ffn_W1.shape[0]
    BT = B * T

    x = tokens.reshape(BT, L, D)
    full = lambda shape: pl.BlockSpec(shape, lambda i: (0,) * len(shape))
    tokens_out = pl.pallas_call(
        _transformer_kernel,
        grid=(BT,),
        in_specs=[
            pl.BlockSpec((1, L, D), lambda i: (i, 0, 0)),
            full((D, 3 * D)), full((1, 3 * D)),
            full((D, D)), full((1, D)),
            full((D, FF)), full((1, FF)),
            full((FF, D)), full((1, D)),
            full((1, D)), full((1, D)), full((1, D)), full((1, D)),
        ],
        out_specs=pl.BlockSpec((1, L, D), lambda i: (i, 0, 0)),
        out_shape=jax.ShapeDtypeStruct((BT, L, D), jnp.float32),
    )(x, Win.T, b_in.reshape(1, -1), Wout.T, b_out.reshape(1, -1),
      ffn_W1.T, ffn_b1.reshape(1, -1), ffn_W2.T, ffn_b2.reshape(1, -1),
      ln1_g.reshape(1, -1), ln1_b.reshape(1, -1),
      ln2_g.reshape(1, -1), ln2_b.reshape(1, -1))

    mem_kernel = functools.partial(_memory_kernel, B=B, T=T, S=S, D=D)
    new_mem = pl.pallas_call(
        mem_kernel,
        in_specs=[pl.BlockSpec(memory_space=pltpu.SMEM)]
        + [pl.BlockSpec(memory_space=pltpu.VMEM)] * 10,
        out_specs=pl.BlockSpec(memory_space=pltpu.VMEM),
        out_shape=jax.ShapeDtypeStruct((B * S, D), jnp.float32),
        scratch_shapes=[pltpu.VMEM((BT, 3 * D), jnp.float32)],
    )(speaker_ids.astype(jnp.int32), edu_reps.reshape(BT, D),
      speaker_memory.reshape(B * S, D),
      Wsq.T, Wsk.T, Wcq.T, Wck.T, Wih.T, Whh.T,
      bih.reshape(1, -1), bhh.reshape(1, -1))

    return tokens_out.reshape(B, T, L, D), new_mem.reshape(B, S, D)


# speedup vs baseline: 7.1426x; 2.0793x over previous
import functools

import jax
import jax.numpy as jnp
from jax.experimental import pallas as pl
from jax.experimental.pallas import tpu as pltpu

N_HEADS = 12
TOP_K = 5
EPS = 1e-5
NEG = -1e30


def _ln(x, g, b):
    mu = jnp.mean(x, axis=-1, keepdims=True)
    var = jnp.mean((x - mu) ** 2, axis=-1, keepdims=True)
    return (x - mu) / jnp.sqrt(var + EPS) * g + b


def _sigmoid(x):
    return 1.0 / (1.0 + jnp.exp(-x))


def _dotT(a, b):
    return jax.lax.dot_general(a, b, (((1,), (1,)), ((), ())),
                               preferred_element_type=jnp.float32)


def _transformer_kernel(x_ref, Win_ref, b_in_ref, Wout_ref, b_out_ref,
                        W1_ref, b1_ref, W2_ref, b2_ref,
                        ln1_g_ref, ln1_b_ref, ln2_g_ref, ln2_b_ref, o_ref,
                        *, BLK):
    BLKL, D = x_ref.shape[0] * x_ref.shape[1], x_ref.shape[2]
    L = x_ref.shape[1]
    x = x_ref[...].reshape(BLKL, D)
    dh = D // N_HEADS
    qkv = jnp.dot(x.astype(jnp.bfloat16), Win_ref[...],
                  preferred_element_type=jnp.float32)
    qkv = qkv + b_in_ref[...]
    scale = 1.0 / (dh ** 0.5)
    heads = []
    qkv_b = qkv.astype(jnp.bfloat16)
    qkvs = []
    for s0 in range(BLK):
        r = slice(s0 * L, (s0 + 1) * L)
        for h in range(N_HEADS):
            qkvs.append((qkv_b[r, h * dh:(h + 1) * dh],
                         qkv_b[r, D + h * dh:D + (h + 1) * dh],
                         qkv_b[r, 2 * D + h * dh:2 * D + (h + 1) * dh]))
    es = [jnp.exp(_dotT(q, k) * scale) for q, k, _ in qkvs]
    evs = [jnp.dot(e.astype(jnp.bfloat16), v,
                   preferred_element_type=jnp.float32)
           for e, (_, _, v) in zip(es, qkvs)]
    rss = [jnp.sum(e, axis=-1, keepdims=True) for e in es]
    heads = [ev / rs for ev, rs in zip(evs, rss)]
    attn = jnp.concatenate(
        [jnp.concatenate(heads[s0 * N_HEADS:(s0 + 1) * N_HEADS], axis=1)
         for s0 in range(BLK)], axis=0)
    attn = jnp.dot(attn.astype(jnp.bfloat16), Wout_ref[...],
                   preferred_element_type=jnp.float32)
    attn = attn + b_out_ref[...]
    tv = _ln(x + attn, ln1_g_ref[...], ln1_b_ref[...])
    tv_b = tv.astype(jnp.bfloat16)
    FF = W1_ref.shape[1]
    NCH = 4
    CH = FF // NCH
    h2 = None
    for c in range(NCH):
        h1 = jnp.dot(tv_b, W1_ref[:, c * CH:(c + 1) * CH],
                     preferred_element_type=jnp.float32)
        h1 = h1 + b1_ref[:, c * CH:(c + 1) * CH]
        h1 = h1 * 0.5 * (1.0 + jax.lax.erf(h1 * (2.0 ** -0.5)))
        part = jnp.dot(h1.astype(jnp.bfloat16), W2_ref[c * CH:(c + 1) * CH, :],
                       preferred_element_type=jnp.float32)
        h2 = part if h2 is None else h2 + part
    h2 = h2 + b2_ref[...]
    o_ref[...] = _ln(tv + h2, ln2_g_ref[...],
                     ln2_b_ref[...]).reshape(BLK, L, D)


def _memory_kernel(ids_ref, edu_ref, mem_ref,
                   WsqT_ref, WskT_ref, WcqT_ref, WckT_ref,
                   WihT_ref, WhhT_ref, bih_ref, bhh_ref,
                   omem_ref, gi_ref,
                   *, B, T, S, D):
    edu = edu_ref[...]
    Ks = jnp.dot(edu, WskT_ref[...], preferred_element_type=jnp.float32)
    Qs = jnp.dot(edu, WsqT_ref[...], preferred_element_type=jnp.float32)
    Kc = jnp.dot(edu, WckT_ref[...], preferred_element_type=jnp.float32)
    Qc = jnp.dot(edu, WcqT_ref[...], preferred_element_type=jnp.float32)

    jidx = jax.lax.broadcasted_iota(jnp.int32, (T, T), 1)
    tidx = jax.lax.broadcasted_iota(jnp.int32, (T, T), 0)
    kk = jnp.minimum(jnp.float32(TOP_K), tidx[:, :1].astype(jnp.float32))
    kk_div = jnp.maximum(kk, 1.0)

    summaries = []
    for b in range(B):
        ks = Ks[b * T:(b + 1) * T]
        qs = Qs[b * T:(b + 1) * T]
        kc = Kc[b * T:(b + 1) * T]
        qc = Qc[b * T:(b + 1) * T]
        sc = (jax.lax.dot_general(qs, ks, (((1,), (1,)), ((), ())),
                                  preferred_element_type=jnp.float32)
              + jax.lax.dot_general(qc, kc, (((1,), (1,)), ((), ())),
                                    preferred_element_type=jnp.float32))
        cur = jnp.where(jidx < tidx, sc, NEG)
        w = jnp.zeros((T, T), jnp.float32)
        for i in range(TOP_K):
            m = jnp.max(cur, axis=1, keepdims=True)
            eq = cur == m
            jsel = jnp.min(jnp.where(eq, jidx, T), axis=1, keepdims=True)
            first = jidx == jsel
            take = first & (jnp.float32(i) < kk)
            w = w + jnp.where(take, 1.0, 0.0)
            cur = jnp.where(first, NEG, cur)
        summaries.append(jnp.dot(w, edu[b * T:(b + 1) * T],
                                 preferred_element_type=jnp.float32) / kk_div)
    summ = jnp.concatenate(summaries, axis=0)
    gi = jnp.dot(summ, WihT_ref[...], preferred_element_type=jnp.float32)
    gi_ref[...] = gi + bih_ref[...]

    omem_ref[...] = mem_ref[...]

    WhhT = WhhT_ref[...]
    bhh = bhh_ref[...]

    def step(t, _):
        hs = []
        for b in range(B):
            s = ids_ref[b, t]
            hs.append(omem_ref[pl.ds(b * S + s, 1), :])
        H = jnp.concatenate(hs, axis=0)
        gh = jnp.dot(H, WhhT, preferred_element_type=jnp.float32) + bhh
        gis = []
        for b in range(B):
            gis.append(gi_ref[pl.ds(b * T + t, 1), :])
        gib = jnp.concatenate(gis, axis=0)
        r = _sigmoid(gib[:, :D] + gh[:, :D])
        z = _sigmoid(gib[:, D:2 * D] + gh[:, D:2 * D])
        n = jnp.tanh(gib[:, 2 * D:] + r * gh[:, 2 * D:])
        newH = (1.0 - z) * n + z * H
        for b in range(B):
            s = ids_ref[b, t]
            omem_ref[pl.ds(b * S + s, 1), :] = newH[b:b + 1, :]
        return 0

    jax.lax.fori_loop(1, T, step, 0)


def kernel(tokens, attn_mask, edu_reps, speaker_ids, speaker_memory,
           Win, b_in, Wout, b_out, ffn_W1, ffn_b1, ffn_W2, ffn_b2,
           ln1_g, ln1_b, ln2_g, ln2_b, Wsq, Wsk, Wcq, Wck,
           Wih, Whh, bih, bhh):
    B, T, L, D = tokens.shape
    S = speaker_memory.shape[1]
    FF = ffn_W1.shape[0]
    BT = B * T

    BLK = 4
    x = tokens.reshape(BT, L, D)
    full = lambda shape: pl.BlockSpec(shape, lambda i: (0,) * len(shape))
    tokens_out = pl.pallas_call(
        functools.partial(_transformer_kernel, BLK=BLK),
        grid=(BT // BLK,),
        in_specs=[
            pl.BlockSpec((BLK, L, D), lambda i: (i, 0, 0)),
            full((D, 3 * D)), full((1, 3 * D)),
            full((D, D)), full((1, D)),
            full((D, FF)), full((1, FF)),
            full((FF, D)), full((1, D)),
            full((1, D)), full((1, D)), full((1, D)), full((1, D)),
        ],
        out_specs=pl.BlockSpec((BLK, L, D), lambda i: (i, 0, 0)),
        out_shape=jax.ShapeDtypeStruct((BT, L, D), jnp.float32),
    )(x, Win.T.astype(jnp.bfloat16), b_in.reshape(1, -1),
      Wout.T.astype(jnp.bfloat16), b_out.reshape(1, -1),
      ffn_W1.T.astype(jnp.bfloat16), ffn_b1.reshape(1, -1),
      ffn_W2.T.astype(jnp.bfloat16), ffn_b2.reshape(1, -1),
      ln1_g.reshape(1, -1), ln1_b.reshape(1, -1),
      ln2_g.reshape(1, -1), ln2_b.reshape(1, -1))

    mem_kernel = functools.partial(_memory_kernel, B=B, T=T, S=S, D=D)
    new_mem = pl.pallas_call(
        mem_kernel,
        in_specs=[pl.BlockSpec(memory_space=pltpu.SMEM)]
        + [pl.BlockSpec(memory_space=pltpu.VMEM)] * 10,
        out_specs=pl.BlockSpec(memory_space=pltpu.VMEM),
        out_shape=jax.ShapeDtypeStruct((B * S, D), jnp.float32),
        scratch_shapes=[pltpu.VMEM((BT, 3 * D), jnp.float32)],
    )(speaker_ids.astype(jnp.int32), edu_reps.reshape(BT, D),
      speaker_memory.reshape(B * S, D),
      Wsq.T, Wsk.T, Wcq.T, Wck.T, Wih.T, Whh.T,
      bih.reshape(1, -1), bhh.reshape(1, -1))

    return tokens_out.reshape(B, T, L, D), new_mem.reshape(B, S, D)


# speedup vs baseline: 7.1591x; 1.0023x over previous
import functools

import jax
import jax.numpy as jnp
from jax import lax
from jax.experimental import pallas as pl
from jax.experimental.pallas import tpu as pltpu
from jax.experimental.pallas import tpu_sc as plsc

N_HEADS = 12
TOP_K = 5
EPS = 1e-5
NEG = -1e30
NEG_SC = -3.0e38


def _ln(x, g, b):
    mu = jnp.mean(x, axis=-1, keepdims=True)
    var = jnp.mean((x - mu) ** 2, axis=-1, keepdims=True)
    return (x - mu) / jnp.sqrt(var + EPS) * g + b


def _sigmoid(x):
    return 1.0 / (1.0 + jnp.exp(-x))


def _dotT(a, b):
    return jax.lax.dot_general(a, b, (((1,), (1,)), ((), ())),
                               preferred_element_type=jnp.float32)


def _transformer_kernel(x_ref, Win_ref, b_in_ref, Wout_ref, b_out_ref,
                        W1_ref, b1_ref, W2_ref, b2_ref,
                        ln1_g_ref, ln1_b_ref, ln2_g_ref, ln2_b_ref, o_ref,
                        *, BLK):
    BLKL, D = x_ref.shape[0] * x_ref.shape[1], x_ref.shape[2]
    L = x_ref.shape[1]
    x = x_ref[...].reshape(BLKL, D)
    dh = D // N_HEADS
    qkv = jnp.dot(x.astype(jnp.bfloat16), Win_ref[...],
                  preferred_element_type=jnp.float32)
    qkv = qkv + b_in_ref[...]
    scale = 1.0 / (dh ** 0.5)
    qkv_b = qkv.astype(jnp.bfloat16)
    qkvs = []
    for s0 in range(BLK):
        r = slice(s0 * L, (s0 + 1) * L)
        for h in range(N_HEADS):
            qkvs.append((qkv_b[r, h * dh:(h + 1) * dh],
                         qkv_b[r, D + h * dh:D + (h + 1) * dh],
                         qkv_b[r, 2 * D + h * dh:2 * D + (h + 1) * dh]))
    es = [jnp.exp(_dotT(q, k) * scale) for q, k, _ in qkvs]
    evs = [jnp.dot(e.astype(jnp.bfloat16), v,
                   preferred_element_type=jnp.float32)
           for e, (_, _, v) in zip(es, qkvs)]
    rss = [jnp.sum(e, axis=-1, keepdims=True) for e in es]
    heads = [ev / rs for ev, rs in zip(evs, rss)]
    attn = jnp.concatenate(
        [jnp.concatenate(heads[s0 * N_HEADS:(s0 + 1) * N_HEADS], axis=1)
         for s0 in range(BLK)], axis=0)
    attn = jnp.dot(attn.astype(jnp.bfloat16), Wout_ref[...],
                   preferred_element_type=jnp.float32)
    attn = attn + b_out_ref[...]
    tv = _ln(x + attn, ln1_g_ref[...], ln1_b_ref[...])
    tv_b = tv.astype(jnp.bfloat16)
    FF = W1_ref.shape[1]
    NCH = 4
    CH = FF // NCH
    h2 = None
    for c in range(NCH):
        h1 = jnp.dot(tv_b, W1_ref[:, c * CH:(c + 1) * CH],
                     preferred_element_type=jnp.float32)
        h1 = h1 + b1_ref[:, c * CH:(c + 1) * CH]
        h1 = h1 * 0.5 * (1.0 + jax.lax.erf(h1 * (2.0 ** -0.5)))
        part = jnp.dot(h1.astype(jnp.bfloat16), W2_ref[c * CH:(c + 1) * CH, :],
                       preferred_element_type=jnp.float32)
        h2 = part if h2 is None else h2 + part
    h2 = h2 + b2_ref[...]
    o_ref[...] = _ln(tv + h2, ln2_g_ref[...],
                     ln2_b_ref[...]).reshape(BLK, L, D)


def _score_kernel(edu_ref, WsqT_ref, WskT_ref, WcqT_ref, WckT_ref, sc_ref,
                  *, B, T):
    edu = edu_ref[...]
    Ks = jnp.dot(edu, WskT_ref[...], preferred_element_type=jnp.float32)
    Qs = jnp.dot(edu, WsqT_ref[...], preferred_element_type=jnp.float32)
    Kc = jnp.dot(edu, WckT_ref[...], preferred_element_type=jnp.float32)
    Qc = jnp.dot(edu, WcqT_ref[...], preferred_element_type=jnp.float32)
    for b in range(B):
        ks = Ks[b * T:(b + 1) * T]
        qs = Qs[b * T:(b + 1) * T]
        kc = Kc[b * T:(b + 1) * T]
        qc = Qc[b * T:(b + 1) * T]
        sc_ref[b] = (_dotT(qs, ks) + _dotT(qc, kc))


def _select_kernel(scores_hbm, edu_hbm, out_hbm, srow_v, rows_v, out_v, sem,
                   *, T, D, TASKS_PER_W):
    wid = lax.axis_index("s") * 2 + lax.axis_index("c")
    lanes = lax.broadcasted_iota(jnp.int32, (16,), 0)
    for i in range(TASKS_PER_W):
        task = wid * TASKS_PER_W + i
        b = task // T
        t = task % T
        pltpu.sync_copy(scores_hbm.at[b, t], srow_v)
        tv16 = jnp.full((16,), t, jnp.int32)
        s0 = jnp.where(lanes < tv16, srow_v[pl.ds(0, 16)], NEG_SC)
        s1 = jnp.where(lanes + 16 < tv16, srow_v[pl.ds(16, 16)], NEG_SC)
        kk = jnp.minimum(TOP_K, t)
        k0, v0 = plsc.sort_key_val(s0, lanes, descending=True)
        k1, v1 = plsc.sort_key_val(s1, lanes + 16, descending=True)
        half = lanes < 8
        cand_k = jnp.where(half, k0, jnp.flip(k1))
        cand_v = jnp.where(half, v0, jnp.flip(v1))
        fk, fv = plsc.sort_key_val(cand_k, cand_v, descending=True)
        weights = [jnp.where(pick < kk, 1.0, 0.0).astype(jnp.float32)
                   for pick in range(TOP_K)]
        row_idx = b * T + fv
        pltpu.async_copy(edu_hbm.at[row_idx], rows_v, sem).wait()
        kk_f = jnp.maximum(jnp.float32(1.0), kk.astype(jnp.float32))
        for c in range(D // 16):
            acc = rows_v[0, pl.ds(c * 16, 16)] * weights[0]
            for p in range(1, TOP_K):
                acc = acc + rows_v[p, pl.ds(c * 16, 16)] * weights[p]
            out_v[pl.ds(c * 16, 16)] = acc / kk_f
        pltpu.sync_copy(out_v, out_hbm.at[task])


def _chain_kernel(ids_ref, summ_ref, mem_ref, WihT_ref, WhhT_ref,
                  bih_ref, bhh_ref, omem_ref, gi_ref, *, B, T, S, D):
    gi = jnp.dot(summ_ref[...], WihT_ref[...],
                 preferred_element_type=jnp.float32)
    gi_ref[...] = gi + bih_ref[...]
    omem_ref[...] = mem_ref[...]
    WhhT = WhhT_ref[...]
    bhh = bhh_ref[...]

    def step(t, _):
        hs = []
        for b in range(B):
            s = ids_ref[b, t]
            hs.append(omem_ref[pl.ds(b * S + s, 1), :])
        H = jnp.concatenate(hs, axis=0)
        gh = jnp.dot(H, WhhT, preferred_element_type=jnp.float32) + bhh
        gis = []
        for b in range(B):
            gis.append(gi_ref[pl.ds(b * T + t, 1), :])
        gib = jnp.concatenate(gis, axis=0)
        r = _sigmoid(gib[:, :D] + gh[:, :D])
        z = _sigmoid(gib[:, D:2 * D] + gh[:, D:2 * D])
        n = jnp.tanh(gib[:, 2 * D:] + r * gh[:, 2 * D:])
        newH = (1.0 - z) * n + z * H
        for b in range(B):
            s = ids_ref[b, t]
            omem_ref[pl.ds(b * S + s, 1), :] = newH[b:b + 1, :]
        return 0

    jax.lax.fori_loop(1, T, step, 0)


def kernel(tokens, attn_mask, edu_reps, speaker_ids, speaker_memory,
           Win, b_in, Wout, b_out, ffn_W1, ffn_b1, ffn_W2, ffn_b2,
           ln1_g, ln1_b, ln2_g, ln2_b, Wsq, Wsk, Wcq, Wck,
           Wih, Whh, bih, bhh):
    B, T, L, D = tokens.shape
    S = speaker_memory.shape[1]
    FF = ffn_W1.shape[0]
    BT = B * T

    BLK = 4
    x = tokens.reshape(BT, L, D)
    full = lambda shape: pl.BlockSpec(shape, lambda i: (0,) * len(shape))
    tokens_out = pl.pallas_call(
        functools.partial(_transformer_kernel, BLK=BLK),
        grid=(BT // BLK,),
        in_specs=[
            pl.BlockSpec((BLK, L, D), lambda i: (i, 0, 0)),
            full((D, 3 * D)), full((1, 3 * D)),
            full((D, D)), full((1, D)),
            full((D, FF)), full((1, FF)),
            full((FF, D)), full((1, D)),
            full((1, D)), full((1, D)), full((1, D)), full((1, D)),
        ],
        out_specs=pl.BlockSpec((BLK, L, D), lambda i: (i, 0, 0)),
        out_shape=jax.ShapeDtypeStruct((BT, L, D), jnp.float32),
    )(x, Win.T.astype(jnp.bfloat16), b_in.reshape(1, -1),
      Wout.T.astype(jnp.bfloat16), b_out.reshape(1, -1),
      ffn_W1.T.astype(jnp.bfloat16), ffn_b1.reshape(1, -1),
      ffn_W2.T.astype(jnp.bfloat16), ffn_b2.reshape(1, -1),
      ln1_g.reshape(1, -1), ln1_b.reshape(1, -1),
      ln2_g.reshape(1, -1), ln2_b.reshape(1, -1))

    edu2 = edu_reps.reshape(BT, D)
    scores = pl.pallas_call(
        functools.partial(_score_kernel, B=B, T=T),
        in_specs=[pl.BlockSpec(memory_space=pltpu.VMEM)] * 5,
        out_specs=pl.BlockSpec(memory_space=pltpu.VMEM),
        out_shape=jax.ShapeDtypeStruct((B, T, T), jnp.float32),
    )(edu2, Wsq.T, Wsk.T, Wcq.T, Wck.T)

    mesh = plsc.VectorSubcoreMesh(core_axis_name="c", subcore_axis_name="s")
    summaries = pl.kernel(
        functools.partial(_select_kernel, T=T, D=D, TASKS_PER_W=BT // 32),
        out_type=jax.ShapeDtypeStruct((BT, D), jnp.float32),
        mesh=mesh,
        scratch_types=[
            pltpu.VMEM((T,), jnp.float32),
            pltpu.VMEM((16, D), jnp.float32),
            pltpu.VMEM((D,), jnp.float32),
            pltpu.SemaphoreType.DMA,
        ],
        compiler_params=pltpu.CompilerParams(needs_layout_passes=False),
    )(scores, edu2)

    new_mem = pl.pallas_call(
        functools.partial(_chain_kernel, B=B, T=T, S=S, D=D),
        in_specs=[pl.BlockSpec(memory_space=pltpu.SMEM)]
        + [pl.BlockSpec(memory_space=pltpu.VMEM)] * 6,
        out_specs=pl.BlockSpec(memory_space=pltpu.VMEM),
        out_shape=jax.ShapeDtypeStruct((B * S, D), jnp.float32),
        scratch_shapes=[pltpu.VMEM((BT, 3 * D), jnp.float32)],
    )(speaker_ids.astype(jnp.int32), summaries,
      speaker_memory.reshape(B * S, D),
      Wih.T, Whh.T, bih.reshape(1, -1), bhh.reshape(1, -1))

    return tokens_out.reshape(B, T, L, D), new_mem.reshape(B, S, D)
